# baseline (speedup 1.0000x reference)
import functools

import jax
import jax.numpy as jnp
from jax import lax
from jax.experimental import pallas as pl
from jax.experimental.pallas import tpu as pltpu
from jax.experimental.pallas import tpu_sc as plsc

N = 10000
E = 320000
D = 128
ED = 16
NLAYER = 3
G = 64

NTILES = 32
NPT = 320
NPAD = NTILES * NPT
CSEG = 1280
NCHUNK = E // CSEG
BROWS = 64
NBUF = 2
EPT = E // NTILES
CG = 400
F32 = jnp.float32
I32 = jnp.int32

_mesh = plsc.VectorSubcoreMesh(
    core_axis_name="c", subcore_axis_name="s", num_cores=2, num_subcores=16
)


def _wid():
    return lax.axis_index("s") * 2 + lax.axis_index("c")


@functools.partial(
    pl.kernel,
    out_type=(
        jax.ShapeDtypeStruct((NPAD, D), F32),
        jax.ShapeDtypeStruct((NPAD, D), F32),
    ),
    mesh=_mesh,
    compiler_params=pltpu.CompilerParams(needs_layout_passes=False),
    scratch_types=[
        pltpu.VMEM((NPT, D), F32),
        pltpu.VMEM((NPT, D), F32),
        pltpu.VMEM((BROWS, D), F32),
        pltpu.VMEM((BROWS, D), F32),
        pltpu.VMEM((CSEG,), I32),
        pltpu.VMEM((CSEG,), I32),
        pltpu.VMEM((CSEG + 16,), I32),
        pltpu.VMEM((CSEG + 16,), I32),
        pltpu.VMEM_SHARED((CSEG, D), F32),
        pltpu.VMEM_SHARED((CSEG, D), F32),
        pltpu.SemaphoreType.DMA,
        pltpu.SemaphoreType.DMA,
        pltpu.SemaphoreType.DMA,
        pltpu.SemaphoreType.DMA,
        pltpu.SemaphoreType.DMA,
        pltpu.SemaphoreType.DMA,
    ],
)
def _seg_sum_max(he_hbm, dst_hbm, sum_hbm, max_hbm,
                 s_ref, m_ref, r0, r1, db0, db1, cidx, cdst, sp0, sp1,
                 g0, g1, ds0, ds1, ss0, ss1):
    rows = (r0, r1)
    gsems = (g0, g1)
    dbufs = (db0, db1)
    dsems = (ds0, ds1)
    spbufs = (sp0, sp1)
    spsems = (ss0, ss1)
    SLICE = CSEG // 16
    wid = _wid()
    sid = lax.axis_index("s")
    lo = wid * NPT
    zf = jnp.zeros((16,), F32)
    zi = jnp.zeros((16,), I32)
    iota = lax.iota(I32, 16)

    def zrow(r, carry):
        for k in range(D // 16):
            s_ref[r, pl.ds(k * 16, 16)] = zf
            m_ref[r, pl.ds(k * 16, 16)] = zf
        return carry

    lax.fori_loop(0, NPT, zrow, 0)

    def zc(i, carry):
        cidx[pl.ds(i * 16, 16)] = zi
        return carry

    lax.fori_loop(0, (CSEG + 16) // 16, zc, 0)

    def issue_dst(c, k):
        pltpu.async_copy(dst_hbm.at[pl.ds(c * CSEG, CSEG)], dbufs[k], dsems[k])

    def wait_dst(c, k):
        pltpu.make_async_copy(
            dst_hbm.at[pl.ds(c * CSEG, CSEG)], dbufs[k], dsems[k]
        ).wait()

    def issue_sp(c, k):
        pltpu.async_copy(
            he_hbm.at[pl.ds(c * CSEG + sid * SLICE, SLICE)],
            spbufs[k].at[pl.ds(sid * SLICE, SLICE)],
            spsems[k],
        )

    def wait_sp(c, k):
        pltpu.make_async_copy(
            he_hbm.at[pl.ds(c * CSEG + sid * SLICE, SLICE)],
            spbufs[k].at[pl.ds(sid * SLICE, SLICE)],
            spsems[k],
        ).wait()

    issue_dst(0, 0)
    issue_dst(1, 1)
    issue_sp(0, 0)
    issue_sp(1, 1)

    def chunk2(g, carry):
        for k in range(2):
            c = g * 2 + k
            wait_dst(c, k)
            wait_sp(c, k)
            plsc.subcore_barrier()

            def scan(i, cnt, k=k):
                v = dbufs[k][pl.ds(i * 16, 16)]
                lv = v - lo
                lu = plsc.bitcast(lv, jnp.uint32)
                msk = lu < jnp.uint32(NPT)
                eid = i * 16 + iota
                plsc.store_compressed(cidx.at[pl.ds(cnt, 16)], eid, mask=msk)
                plsc.store_compressed(cdst.at[pl.ds(cnt, 16)], lv, mask=msk)
                return cnt + plsc.all_reduce_population_count(msk)[0]

            cnt = lax.fori_loop(0, CSEG // 16, scan, 0)

            def issue_g(b, r, k=k):
                pltpu.async_copy(
                    spbufs[k].at[cidx.at[pl.ds(b * BROWS, BROWS)]],
                    rows[r], gsems[r],
                )

            def wait_g(b, r, k=k):
                pltpu.make_async_copy(
                    spbufs[k].at[cidx.at[pl.ds(b * BROWS, BROWS)]],
                    rows[r], gsems[r],
                ).wait()

            nb = (cnt + BROWS - 1) // BROWS
            for r in range(NBUF):
                @pl.when(r < nb)
                def _(r=r):
                    issue_g(r, r)

            def grp(gg, cr):
                for r in range(NBUF):
                    b = gg * NBUF + r

                    @pl.when(b < nb)
                    def _(b=b, r=r):
                        wait_g(b, r)
                        m = jnp.minimum(cnt - b * BROWS, BROWS)

                        def upd(jj, cr2):
                            d = cdst[pl.ds(b * BROWS + jj, 16)][0]
                            for kk in range(D // 16):
                                sl = pl.ds(kk * 16, 16)
                                rv = rows[r][jj, sl]
                                s_ref[d, sl] = s_ref[d, sl] + rv
                                m_ref[d, sl] = jnp.maximum(m_ref[d, sl], rv)
                            return cr2

                        lax.fori_loop(0, m, upd, 0)

                        @pl.when(b + NBUF < nb)
                        def _():
                            issue_g(b + NBUF, r)

                return cr

            lax.fori_loop(0, (nb + NBUF - 1) // NBUF, grp, 0)
            plsc.subcore_barrier()

            @pl.when(c + 2 < NCHUNK)
            def _(c=c, k=k):
                issue_dst(c + 2, k)
                issue_sp(c + 2, k)

        return carry

    lax.fori_loop(0, NCHUNK // 2, chunk2, 0)
    pltpu.sync_copy(s_ref, sum_hbm.at[pl.ds(lo, NPT)])
    pltpu.sync_copy(m_ref, max_hbm.at[pl.ds(lo, NPT)])


NUNIT = E // 128
UPT = NUNIT // NTILES
UREM = NUNIT % NTILES
TROWS = NPAD // 16


@functools.partial(
    pl.kernel,
    out_type=jax.ShapeDtypeStruct((2, NPAD, D), F32),
    mesh=_mesh,
    compiler_params=pltpu.CompilerParams(needs_layout_passes=False),
    scratch_types=[
        pltpu.VMEM_SHARED((NPAD, D), F32),
        pltpu.VMEM((128, D), F32),
        pltpu.VMEM((128, D), F32),
        pltpu.VMEM((1, 128), I32),
        pltpu.VMEM((1, 128), I32),
        pltpu.SemaphoreType.DMA,
        pltpu.SemaphoreType.DMA,
        pltpu.SemaphoreType.DMA,
        pltpu.SemaphoreType.DMA,
    ],
)
def _seg_sum_atomic(he_hbm, dst2d_hbm, out_hbm,
                    table, e0, e1, i0, i1, es0, es1, is0, is1):
    ebufs = (e0, e1)
    ibufs = (i0, i1)
    esems = (es0, es1)
    isems = (is0, is1)
    cid = lax.axis_index("c")
    sid = lax.axis_index("s")
    wid = sid * 2 + cid
    zf = jnp.zeros((16,), F32)

    def zr(r, carry):
        for k in range(D // 16):
            e0[r, pl.ds(k * 16, 16)] = zf
        return carry

    lax.fori_loop(0, 128, zr, 0)
    zlo = sid * TROWS
    for part in range(TROWS // 128):
        pltpu.sync_copy(e0, table.at[pl.ds(zlo + part * 128, 128)])
    plsc.subcore_barrier()

    u0 = wid * UPT + jnp.minimum(wid, UREM)
    nu = UPT + (wid < UREM).astype(I32)

    def issue(t, k):
        u = u0 + t
        pltpu.async_copy(he_hbm.at[pl.ds(u * 128, 128)], ebufs[k], esems[k])
        pltpu.async_copy(dst2d_hbm.at[pl.ds(u, 1)], ibufs[k], isems[k])

    def wait_u(t, k):
        u = u0 + t
        pltpu.make_async_copy(
            he_hbm.at[pl.ds(u * 128, 128)], ebufs[k], esems[k]
        ).wait()
        pltpu.make_async_copy(
            dst2d_hbm.at[pl.ds(u, 1)], ibufs[k], isems[k]
        ).wait()

    for k in range(2):
        @pl.when(k < nu)
        def _(k=k):
            issue(k, k)

    def grp(g, carry):
        for k in range(2):
            t = g * 2 + k

            @pl.when(t < nu)
            def _(t=t, k=k):
                wait_u(t, k)
                pltpu.sync_copy(ebufs[k], table.at[ibufs[k].at[0]], add=True)

                @pl.when(t + 2 < nu)
                def _():
                    issue(t + 2, k)

        return carry

    lax.fori_loop(0, (UPT + 2) // 2, grp, 0)
    plsc.subcore_barrier()
    pltpu.sync_copy(
        table.at[pl.ds(zlo, TROWS)], out_hbm.at[cid].at[pl.ds(zlo, TROWS)]
    )


@functools.partial(
    pl.kernel,
    out_type=jax.ShapeDtypeStruct((E, D), F32),
    mesh=_mesh,
    compiler_params=pltpu.CompilerParams(needs_layout_passes=False),
    scratch_types=[
        pltpu.VMEM((CG,), I32),
        pltpu.VMEM((CG, D), F32),
        pltpu.SemaphoreType.DMA,
    ],
)
def _gather_rows(g_hbm, src_hbm, out_hbm, sbuf, gbuf, sem):
    wid = _wid()

    def chunk(c, carry):
        off = wid * EPT + c * CG
        pltpu.sync_copy(src_hbm.at[pl.ds(off, CG)], sbuf)
        cps = [
            pltpu.async_copy(
                g_hbm.at[sbuf.at[pl.ds(j * 80, 80)]],
                gbuf.at[pl.ds(j * 80, 80)],
                sem,
            )
            for j in range(CG // 80)
        ]
        for cp in cps:
            cp.wait()
        pltpu.sync_copy(gbuf, out_hbm.at[pl.ds(off, CG)])
        return carry

    lax.fori_loop(0, EPT // CG, chunk, 0)


def _relu_mm_body(x_ref, w_ref, o_ref):
    o_ref[...] = jnp.maximum(
        jnp.dot(x_ref[...], w_ref[...], preferred_element_type=F32), 0.0
    )


def _node_init(x, w):
    return pl.pallas_call(
        _relu_mm_body,
        out_shape=jax.ShapeDtypeStruct((N, D), F32),
    )(x, w)


def _edge_init(ea, w):
    blk = 8000
    return pl.pallas_call(
        _relu_mm_body,
        grid=(E // blk,),
        in_specs=[
            pl.BlockSpec((blk, ED), lambda i: (i, 0)),
            pl.BlockSpec((ED, D), lambda i: (0, 0)),
        ],
        out_specs=pl.BlockSpec((blk, D), lambda i: (i, 0)),
        out_shape=jax.ShapeDtypeStruct((E, D), F32),
    )(ea, w)


def _node_update_body(hn_ref, s_ref, m_ref, wc_ref, wh_ref, hn_out, g_out):
    msg = s_ref[...] * m_ref[...]
    hn = jnp.maximum(
        jnp.dot(hn_ref[...] + msg, wc_ref[...], preferred_element_type=F32), 0.0
    )
    hn_out[...] = hn
    g_out[...] = jnp.dot(hn, wh_ref[...], preferred_element_type=F32)


def _node_update(hn, s, m, wc, wh):
    return pl.pallas_call(
        _node_update_body,
        out_shape=(
            jax.ShapeDtypeStruct((N, D), F32),
            jax.ShapeDtypeStruct((N, D), F32),
        ),
    )(hn, s, m, wc, wh)


def _edge_premul_body(he_ref, he0_ref, wh_ref, o_ref):
    o_ref[...] = he0_ref[...] - jnp.dot(
        he_ref[...], wh_ref[...], preferred_element_type=F32
    )


def _edge_premul(he, he0, wh):
    blk = 8000
    return pl.pallas_call(
        _edge_premul_body,
        grid=(E // blk,),
        in_specs=[
            pl.BlockSpec((blk, D), lambda i: (i, 0)),
            pl.BlockSpec((blk, D), lambda i: (i, 0)),
            pl.BlockSpec((D, D), lambda i: (0, 0)),
        ],
        out_specs=pl.BlockSpec((blk, D), lambda i: (i, 0)),
        out_shape=jax.ShapeDtypeStruct((E, D), F32),
    )(he, he0, wh)


def _edge_update_body(he_ref, he0_ref, gs_ref, wh_ref, o_ref):
    o_ref[...] = jnp.maximum(
        he0_ref[...]
        - jnp.dot(he_ref[...], wh_ref[...], preferred_element_type=F32)
        + gs_ref[...],
        0.0,
    )


def _edge_update(he, he0, gs, wh):
    blk = 8000
    return pl.pallas_call(
        _edge_update_body,
        grid=(E // blk,),
        in_specs=[
            pl.BlockSpec((blk, D), lambda i: (i, 0)),
            pl.BlockSpec((blk, D), lambda i: (i, 0)),
            pl.BlockSpec((blk, D), lambda i: (i, 0)),
            pl.BlockSpec((D, D), lambda i: (0, 0)),
        ],
        out_specs=pl.BlockSpec((blk, D), lambda i: (i, 0)),
        out_shape=jax.ShapeDtypeStruct((E, D), F32),
    )(he, he0, gs, wh)


def _readout_body(hn_ref, agg_ref, b_ref, wo_ref, lw_ref, lb_ref, o_ref):
    wo = wo_ref[...]
    agg = agg_ref[0, :N] + agg_ref[1, :N]
    h_atom = jnp.maximum(
        jnp.dot(hn_ref[...], wo[:D], preferred_element_type=F32)
        + jnp.dot(agg, wo[D:], preferred_element_type=F32),
        0.0,
    )
    onehot = (b_ref[...] == lax.broadcasted_iota(I32, (1, G), 1)).astype(F32)
    sums = lax.dot_general(
        onehot, h_atom, (((0,), (0,)), ((), ())), preferred_element_type=F32
    )
    counts = jnp.sum(onehot, axis=0)[:, None]
    h_mol = sums / jnp.maximum(counts, 1.0)
    o_ref[...] = (
        jnp.dot(h_mol, lw_ref[...], preferred_element_type=F32) + lb_ref[...]
    )


def _readout(hn, agg, batch2d, wo, lw, lb2d):
    return pl.pallas_call(
        _readout_body,
        out_shape=jax.ShapeDtypeStruct((G, 1), F32),
    )(hn, agg, batch2d, wo, lw, lb2d)


@jax.jit
def kernel(x, edge_index, edge_attr, batch, W_i_node, W_i_edge, W_comm, W_h,
           W_o, lin_W, lin_b):
    src = edge_index[0]
    dst = edge_index[1]
    hn = _node_init(x, W_i_node)
    he0 = _edge_init(edge_attr, W_i_edge)
    he = he0
    for l in range(NLAYER):
        s_pad, m_pad = _seg_sum_max(he, dst)
        hn, g = _node_update(hn, s_pad[:N], m_pad[:N], W_comm[l], W_h[l])
        gs = _gather_rows(g, src)
        he = _edge_update(he, he0, gs, W_h[l])
    p = _seg_sum_atomic(he, dst.reshape(NUNIT, 128))
    return _readout(
        hn, p, batch.reshape(N, 1), W_o, lin_W, lin_b.reshape(1, 1)
    )


# speedup vs baseline: 1.1676x; 1.1676x over previous
import functools

import jax
import jax.numpy as jnp
from jax import lax
from jax.experimental import pallas as pl
from jax.experimental.pallas import tpu as pltpu
from jax.experimental.pallas import tpu_sc as plsc

N = 10000
E = 320000
D = 128
ED = 16
NLAYER = 3
G = 64

NTILES = 32
NPT = 320
NPAD = NTILES * NPT
CSEG = 3200
NCHUNK = E // CSEG
CAP = 6400
BROWS = 64
NBUF = 3
EPT = E // NTILES
CG = 400
F32 = jnp.float32
I32 = jnp.int32

_mesh = plsc.VectorSubcoreMesh(
    core_axis_name="c", subcore_axis_name="s", num_cores=2, num_subcores=16
)


def _wid():
    return lax.axis_index("s") * 2 + lax.axis_index("c")


@functools.partial(
    pl.kernel,
    out_type=(
        jax.ShapeDtypeStruct((NPAD, D), F32),
        jax.ShapeDtypeStruct((NPAD, D), F32),
    ),
    mesh=_mesh,
    compiler_params=pltpu.CompilerParams(needs_layout_passes=False),
    scratch_types=[
        pltpu.VMEM((NPT, D), F32),
        pltpu.VMEM((NPT, D), F32),
        pltpu.VMEM((BROWS, D), F32),
        pltpu.VMEM((BROWS, D), F32),
        pltpu.VMEM((BROWS, D), F32),
        pltpu.VMEM((CSEG,), I32),
        pltpu.VMEM((CSEG,), I32),
        pltpu.VMEM((CAP + 16,), I32),
        pltpu.VMEM((CAP + 16,), I32),
        pltpu.SMEM((8,), I32),
        pltpu.SemaphoreType.DMA,
        pltpu.SemaphoreType.DMA,
        pltpu.SemaphoreType.DMA,
        pltpu.SemaphoreType.DMA,
        pltpu.SemaphoreType.DMA,
    ],
)
def _seg_sum_max(he_hbm, dst_hbm, sum_hbm, max_hbm,
                 s_ref, m_ref, r0, r1, r2, db0, db1, cidx, cdst, cnt_ref,
                 g0, g1, g2, ds0, ds1):
    rows = (r0, r1, r2)
    gsems = (g0, g1, g2)
    dbufs = (db0, db1)
    dsems = (ds0, ds1)
    wid = _wid()
    lo = wid * NPT
    zf = jnp.zeros((16,), F32)
    zi = jnp.zeros((16,), I32)
    iota = lax.iota(I32, 16)

    def zrow(r, carry):
        for k in range(D // 16):
            s_ref[r, pl.ds(k * 16, 16)] = zf
            m_ref[r, pl.ds(k * 16, 16)] = zf
        return carry

    lax.fori_loop(0, NPT, zrow, 0)

    def zc(i, carry):
        cidx[pl.ds(i * 16, 16)] = zi
        return carry

    lax.fori_loop(0, (CAP + 16) // 16, zc, 0)
    cnt_ref[0] = 0

    def issue_g(b, k):
        pltpu.async_copy(
            he_hbm.at[cidx.at[pl.ds(b * BROWS, BROWS)]], rows[k], gsems[k]
        )

    def wait_g(b, k):
        pltpu.make_async_copy(
            he_hbm.at[cidx.at[pl.ds(b * BROWS, BROWS)]], rows[k], gsems[k]
        ).wait()

    def drain():
        cnt = cnt_ref[0]
        nb = (cnt + BROWS - 1) // BROWS
        for k in range(NBUF):
            @pl.when(k < nb)
            def _(k=k):
                issue_g(k, k)

        def grp(g, carry):
            for k in range(NBUF):
                b = g * NBUF + k

                @pl.when(b < nb)
                def _(b=b, k=k):
                    wait_g(b, k)
                    m = jnp.minimum(cnt - b * BROWS, BROWS)

                    def upd(jj, cr):
                        d = cdst[pl.ds(b * BROWS + jj, 16)][0]
                        for kk in range(D // 16):
                            sl = pl.ds(kk * 16, 16)
                            rv = rows[k][jj, sl]
                            s_ref[d, sl] = s_ref[d, sl] + rv
                            m_ref[d, sl] = jnp.maximum(m_ref[d, sl], rv)
                        return cr

                    lax.fori_loop(0, m, upd, 0)

                    @pl.when(b + NBUF < nb)
                    def _():
                        issue_g(b + NBUF, k)

            return carry

        lax.fori_loop(0, (nb + NBUF - 1) // NBUF, grp, 0)
        cnt_ref[0] = 0

    def issue_dst(c, k):
        pltpu.async_copy(dst_hbm.at[pl.ds(c * CSEG, CSEG)], dbufs[k], dsems[k])

    def wait_dst(c, k):
        pltpu.make_async_copy(
            dst_hbm.at[pl.ds(c * CSEG, CSEG)], dbufs[k], dsems[k]
        ).wait()

    issue_dst(0, 0)
    issue_dst(1, 1)

    def chunk2(g, carry):
        for k in range(2):
            c = g * 2 + k
            wait_dst(c, k)
            cnt0 = cnt_ref[0]

            def scan(i, cnt, k=k, c=c):
                v = dbufs[k][pl.ds(i * 16, 16)]
                lv = v - lo
                lu = plsc.bitcast(lv, jnp.uint32)
                msk = lu < jnp.uint32(NPT)
                eid = c * CSEG + i * 16 + iota
                plsc.store_compressed(cidx.at[pl.ds(cnt, 16)], eid, mask=msk)
                plsc.store_compressed(cdst.at[pl.ds(cnt, 16)], lv, mask=msk)
                return cnt + plsc.all_reduce_population_count(msk)[0]

            cnt1 = lax.fori_loop(0, CSEG // 16, scan, cnt0)
            cnt_ref[0] = cnt1

            @pl.when(c + 2 < NCHUNK)
            def _(c=c, k=k):
                issue_dst(c + 2, k)

            @pl.when(cnt1 > CAP - CSEG)
            def _():
                drain()

        return carry

    lax.fori_loop(0, NCHUNK // 2, chunk2, 0)

    @pl.when(cnt_ref[0] > 0)
    def _():
        drain()

    pltpu.sync_copy(s_ref, sum_hbm.at[pl.ds(lo, NPT)])
    pltpu.sync_copy(m_ref, max_hbm.at[pl.ds(lo, NPT)])


NUNIT = E // 128
UPT = NUNIT // NTILES
UREM = NUNIT % NTILES
TROWS = NPAD // 16


@functools.partial(
    pl.kernel,
    out_type=jax.ShapeDtypeStruct((2, NPAD, D), F32),
    mesh=_mesh,
    compiler_params=pltpu.CompilerParams(needs_layout_passes=False),
    scratch_types=[
        pltpu.VMEM_SHARED((NPAD, D), F32),
        pltpu.VMEM((128, D), F32),
        pltpu.VMEM((128, D), F32),
        pltpu.VMEM((1, 128), I32),
        pltpu.VMEM((1, 128), I32),
        pltpu.SemaphoreType.DMA,
        pltpu.SemaphoreType.DMA,
        pltpu.SemaphoreType.DMA,
        pltpu.SemaphoreType.DMA,
    ],
)
def _seg_sum_atomic(he_hbm, dst2d_hbm, out_hbm,
                    table, e0, e1, i0, i1, es0, es1, is0, is1):
    ebufs = (e0, e1)
    ibufs = (i0, i1)
    esems = (es0, es1)
    isems = (is0, is1)
    cid = lax.axis_index("c")
    sid = lax.axis_index("s")
    wid = sid * 2 + cid
    zf = jnp.zeros((16,), F32)

    def zr(r, carry):
        for k in range(D // 16):
            e0[r, pl.ds(k * 16, 16)] = zf
        return carry

    lax.fori_loop(0, 128, zr, 0)
    zlo = sid * TROWS
    for part in range(TROWS // 128):
        pltpu.sync_copy(e0, table.at[pl.ds(zlo + part * 128, 128)])
    plsc.subcore_barrier()

    u0 = wid * UPT + jnp.minimum(wid, UREM)
    nu = UPT + (wid < UREM).astype(I32)

    def issue(t, k):
        u = u0 + t
        pltpu.async_copy(he_hbm.at[pl.ds(u * 128, 128)], ebufs[k], esems[k])
        pltpu.async_copy(dst2d_hbm.at[pl.ds(u, 1)], ibufs[k], isems[k])

    def wait_u(t, k):
        u = u0 + t
        pltpu.make_async_copy(
            he_hbm.at[pl.ds(u * 128, 128)], ebufs[k], esems[k]
        ).wait()
        pltpu.make_async_copy(
            dst2d_hbm.at[pl.ds(u, 1)], ibufs[k], isems[k]
        ).wait()

    for k in range(2):
        @pl.when(k < nu)
        def _(k=k):
            issue(k, k)

    def grp(g, carry):
        for k in range(2):
            t = g * 2 + k

            @pl.when(t < nu)
            def _(t=t, k=k):
                wait_u(t, k)
                pltpu.sync_copy(ebufs[k], table.at[ibufs[k].at[0]], add=True)

                @pl.when(t + 2 < nu)
                def _():
                    issue(t + 2, k)

        return carry

    lax.fori_loop(0, (UPT + 2) // 2, grp, 0)
    plsc.subcore_barrier()
    pltpu.sync_copy(
        table.at[pl.ds(zlo, TROWS)], out_hbm.at[cid].at[pl.ds(zlo, TROWS)]
    )


@functools.partial(
    pl.kernel,
    out_type=jax.ShapeDtypeStruct((E, D), F32),
    mesh=_mesh,
    compiler_params=pltpu.CompilerParams(needs_layout_passes=False),
    scratch_types=[
        pltpu.VMEM((CG,), I32),
        pltpu.VMEM((CG, D), F32),
        pltpu.SemaphoreType.DMA,
    ],
)
def _gather_rows(g_hbm, src_hbm, out_hbm, sbuf, gbuf, sem):
    wid = _wid()

    def chunk(c, carry):
        off = wid * EPT + c * CG
        pltpu.sync_copy(src_hbm.at[pl.ds(off, CG)], sbuf)
        cps = [
            pltpu.async_copy(
                g_hbm.at[sbuf.at[pl.ds(j * 80, 80)]],
                gbuf.at[pl.ds(j * 80, 80)],
                sem,
            )
            for j in range(CG // 80)
        ]
        for cp in cps:
            cp.wait()
        pltpu.sync_copy(gbuf, out_hbm.at[pl.ds(off, CG)])
        return carry

    lax.fori_loop(0, EPT // CG, chunk, 0)


def _relu_mm_body(x_ref, w_ref, o_ref):
    o_ref[...] = jnp.maximum(
        jnp.dot(x_ref[...], w_ref[...], preferred_element_type=F32), 0.0
    )


def _node_init(x, w):
    return pl.pallas_call(
        _relu_mm_body,
        out_shape=jax.ShapeDtypeStruct((N, D), F32),
    )(x, w)


def _edge_init(ea, w):
    blk = 8000
    return pl.pallas_call(
        _relu_mm_body,
        grid=(E // blk,),
        in_specs=[
            pl.BlockSpec((blk, ED), lambda i: (i, 0)),
            pl.BlockSpec((ED, D), lambda i: (0, 0)),
        ],
        out_specs=pl.BlockSpec((blk, D), lambda i: (i, 0)),
        out_shape=jax.ShapeDtypeStruct((E, D), F32),
    )(ea, w)


def _node_update_body(hn_ref, s_ref, m_ref, wc_ref, wh_ref, hn_out, g_out):
    msg = s_ref[...] * m_ref[...]
    hn = jnp.maximum(
        jnp.dot(hn_ref[...] + msg, wc_ref[...], preferred_element_type=F32), 0.0
    )
    hn_out[...] = hn
    g_out[...] = jnp.dot(hn, wh_ref[...], preferred_element_type=F32)


def _node_update(hn, s, m, wc, wh):
    return pl.pallas_call(
        _node_update_body,
        out_shape=(
            jax.ShapeDtypeStruct((N, D), F32),
            jax.ShapeDtypeStruct((N, D), F32),
        ),
    )(hn, s, m, wc, wh)


def _edge_premul_body(he_ref, he0_ref, wh_ref, o_ref):
    o_ref[...] = he0_ref[...] - jnp.dot(
        he_ref[...], wh_ref[...], preferred_element_type=F32
    )


def _edge_premul(he, he0, wh):
    blk = 8000
    return pl.pallas_call(
        _edge_premul_body,
        grid=(E // blk,),
        in_specs=[
            pl.BlockSpec((blk, D), lambda i: (i, 0)),
            pl.BlockSpec((blk, D), lambda i: (i, 0)),
            pl.BlockSpec((D, D), lambda i: (0, 0)),
        ],
        out_specs=pl.BlockSpec((blk, D), lambda i: (i, 0)),
        out_shape=jax.ShapeDtypeStruct((E, D), F32),
    )(he, he0, wh)


def _edge_update_body(he_ref, he0_ref, gs_ref, wh_ref, o_ref):
    o_ref[...] = jnp.maximum(
        he0_ref[...]
        - jnp.dot(he_ref[...], wh_ref[...], preferred_element_type=F32)
        + gs_ref[...],
        0.0,
    )


def _edge_update(he, he0, gs, wh):
    blk = 8000
    return pl.pallas_call(
        _edge_update_body,
        grid=(E // blk,),
        in_specs=[
            pl.BlockSpec((blk, D), lambda i: (i, 0)),
            pl.BlockSpec((blk, D), lambda i: (i, 0)),
            pl.BlockSpec((blk, D), lambda i: (i, 0)),
            pl.BlockSpec((D, D), lambda i: (0, 0)),
        ],
        out_specs=pl.BlockSpec((blk, D), lambda i: (i, 0)),
        out_shape=jax.ShapeDtypeStruct((E, D), F32),
    )(he, he0, gs, wh)


def _readout_body(hn_ref, agg_ref, b_ref, wo_ref, lw_ref, lb_ref, o_ref):
    wo = wo_ref[...]
    agg = agg_ref[0, :N] + agg_ref[1, :N]
    h_atom = jnp.maximum(
        jnp.dot(hn_ref[...], wo[:D], preferred_element_type=F32)
        + jnp.dot(agg, wo[D:], preferred_element_type=F32),
        0.0,
    )
    onehot = (b_ref[...] == lax.broadcasted_iota(I32, (1, G), 1)).astype(F32)
    sums = lax.dot_general(
        onehot, h_atom, (((0,), (0,)), ((), ())), preferred_element_type=F32
    )
    counts = jnp.sum(onehot, axis=0)[:, None]
    h_mol = sums / jnp.maximum(counts, 1.0)
    o_ref[...] = (
        jnp.dot(h_mol, lw_ref[...], preferred_element_type=F32) + lb_ref[...]
    )


def _readout(hn, agg, batch2d, wo, lw, lb2d):
    return pl.pallas_call(
        _readout_body,
        out_shape=jax.ShapeDtypeStruct((G, 1), F32),
    )(hn, agg, batch2d, wo, lw, lb2d)


@jax.jit
def kernel(x, edge_index, edge_attr, batch, W_i_node, W_i_edge, W_comm, W_h,
           W_o, lin_W, lin_b):
    src = edge_index[0]
    dst = edge_index[1]
    hn = _node_init(x, W_i_node)
    he0 = _edge_init(edge_attr, W_i_edge)
    he = he0
    for l in range(NLAYER):
        s_pad, m_pad = _seg_sum_max(he, dst)
        hn, g = _node_update(hn, s_pad[:N], m_pad[:N], W_comm[l], W_h[l])
        gs = _gather_rows(g, src)
        he = _edge_update(he, he0, gs, W_h[l])
    p = _seg_sum_atomic(he, dst.reshape(NUNIT, 128))
    return _readout(
        hn, p, batch.reshape(N, 1), W_o, lin_W, lin_b.reshape(1, 1)
    )


# speedup vs baseline: 1.1874x; 1.0169x over previous
import functools

import jax
import jax.numpy as jnp
from jax import lax
from jax.experimental import pallas as pl
from jax.experimental.pallas import tpu as pltpu
from jax.experimental.pallas import tpu_sc as plsc

N = 10000
E = 320000
D = 128
ED = 16
NLAYER = 3
G = 64

NTILES = 32
NPT = 320
NPAD = NTILES * NPT
CSEG = 3200
NCHUNK = E // CSEG
CAP = 6400
BROWS = 64
NBUF = 3
EPT = E // NTILES
CG = 400
F32 = jnp.float32
I32 = jnp.int32

_mesh = plsc.VectorSubcoreMesh(
    core_axis_name="c", subcore_axis_name="s", num_cores=2, num_subcores=16
)


def _wid():
    return lax.axis_index("s") * 2 + lax.axis_index("c")


@functools.partial(
    pl.kernel,
    out_type=(
        jax.ShapeDtypeStruct((NPAD, D), F32),
        jax.ShapeDtypeStruct((NPAD, D), F32),
    ),
    mesh=_mesh,
    compiler_params=pltpu.CompilerParams(needs_layout_passes=False),
    scratch_types=[
        pltpu.VMEM((NPT, D), F32),
        pltpu.VMEM((NPT, D), F32),
        pltpu.VMEM((BROWS, D), F32),
        pltpu.VMEM((BROWS, D), F32),
        pltpu.VMEM((BROWS, D), F32),
        pltpu.VMEM((CSEG,), I32),
        pltpu.VMEM((CSEG,), I32),
        pltpu.VMEM((CAP + 16,), I32),
        pltpu.VMEM((CAP + 16,), I32),
        pltpu.SMEM((8,), I32),
        pltpu.SemaphoreType.DMA,
        pltpu.SemaphoreType.DMA,
        pltpu.SemaphoreType.DMA,
        pltpu.SemaphoreType.DMA,
        pltpu.SemaphoreType.DMA,
    ],
)
def _seg_sum_max(he_hbm, dst_hbm, sum_hbm, max_hbm,
                 s_ref, m_ref, r0, r1, r2, db0, db1, cidx, cdst, cnt_ref,
                 g0, g1, g2, ds0, ds1):
    rows = (r0, r1, r2)
    gsems = (g0, g1, g2)
    dbufs = (db0, db1)
    dsems = (ds0, ds1)
    wid = _wid()
    lo = wid * NPT
    zf = jnp.zeros((16,), F32)
    zi = jnp.zeros((16,), I32)
    iota = lax.iota(I32, 16)

    def zrow(r, carry):
        for k in range(D // 16):
            s_ref[r, pl.ds(k * 16, 16)] = zf
            m_ref[r, pl.ds(k * 16, 16)] = zf
        return carry

    lax.fori_loop(0, NPT, zrow, 0)

    def zc(i, carry):
        cidx[pl.ds(i * 16, 16)] = zi
        return carry

    lax.fori_loop(0, (CAP + 16) // 16, zc, 0)
    cnt_ref[0] = 0

    def issue_g(b, k):
        pltpu.async_copy(
            he_hbm.at[cidx.at[pl.ds(b * BROWS, BROWS)]], rows[k], gsems[k]
        )

    def wait_g(b, k):
        pltpu.make_async_copy(
            he_hbm.at[cidx.at[pl.ds(b * BROWS, BROWS)]], rows[k], gsems[k]
        ).wait()

    def drain():
        cnt = cnt_ref[0]
        nb = (cnt + BROWS - 1) // BROWS
        for k in range(NBUF):
            @pl.when(k < nb)
            def _(k=k):
                issue_g(k, k)

        def grp(g, carry):
            for k in range(NBUF):
                b = g * NBUF + k

                @pl.when(b < nb)
                def _(b=b, k=k):
                    wait_g(b, k)
                    m = jnp.minimum(cnt - b * BROWS, BROWS)

                    def upd(jj, cr):
                        d = cdst[pl.ds(b * BROWS + jj, 16)][0]
                        for kk in range(D // 16):
                            sl = pl.ds(kk * 16, 16)
                            rv = rows[k][jj, sl]
                            s_ref[d, sl] = s_ref[d, sl] + rv
                            m_ref[d, sl] = jnp.maximum(m_ref[d, sl], rv)
                        return cr

                    lax.fori_loop(0, m, upd, 0)

                    @pl.when(b + NBUF < nb)
                    def _():
                        issue_g(b + NBUF, k)

            return carry

        lax.fori_loop(0, (nb + NBUF - 1) // NBUF, grp, 0)
        cnt_ref[0] = 0

    def issue_dst(c, k):
        pltpu.async_copy(dst_hbm.at[pl.ds(c * CSEG, CSEG)], dbufs[k], dsems[k])

    def wait_dst(c, k):
        pltpu.make_async_copy(
            dst_hbm.at[pl.ds(c * CSEG, CSEG)], dbufs[k], dsems[k]
        ).wait()

    issue_dst(0, 0)
    issue_dst(1, 1)

    def chunk2(g, carry):
        for k in range(2):
            c = g * 2 + k
            wait_dst(c, k)
            cnt0 = cnt_ref[0]

            def scan(i, cnt, k=k, c=c):
                v = dbufs[k][pl.ds(i * 16, 16)]
                lv = v - lo
                lu = plsc.bitcast(lv, jnp.uint32)
                msk = lu < jnp.uint32(NPT)
                eid = c * CSEG + i * 16 + iota
                plsc.store_compressed(cidx.at[pl.ds(cnt, 16)], eid, mask=msk)
                plsc.store_compressed(cdst.at[pl.ds(cnt, 16)], lv, mask=msk)
                return cnt + plsc.all_reduce_population_count(msk)[0]

            cnt1 = lax.fori_loop(0, CSEG // 16, scan, cnt0)
            cnt_ref[0] = cnt1

            @pl.when(c + 2 < NCHUNK)
            def _(c=c, k=k):
                issue_dst(c + 2, k)

            @pl.when(cnt1 > CAP - CSEG)
            def _():
                drain()

        return carry

    lax.fori_loop(0, NCHUNK // 2, chunk2, 0)

    @pl.when(cnt_ref[0] > 0)
    def _():
        drain()

    pltpu.sync_copy(s_ref, sum_hbm.at[pl.ds(lo, NPT)])
    pltpu.sync_copy(m_ref, max_hbm.at[pl.ds(lo, NPT)])


NUNIT = E // 128
UPT = NUNIT // NTILES
UREM = NUNIT % NTILES
TROWS = NPAD // 16


@functools.partial(
    pl.kernel,
    out_type=jax.ShapeDtypeStruct((2, NPAD, D), F32),
    mesh=_mesh,
    compiler_params=pltpu.CompilerParams(needs_layout_passes=False),
    scratch_types=[
        pltpu.VMEM_SHARED((NPAD, D), F32),
        pltpu.VMEM((128, D), F32),
        pltpu.VMEM((128, D), F32),
        pltpu.VMEM((1, 128), I32),
        pltpu.VMEM((1, 128), I32),
        pltpu.SemaphoreType.DMA,
        pltpu.SemaphoreType.DMA,
        pltpu.SemaphoreType.DMA,
        pltpu.SemaphoreType.DMA,
    ],
)
def _seg_sum_atomic(he_hbm, dst2d_hbm, out_hbm,
                    table, e0, e1, i0, i1, es0, es1, is0, is1):
    ebufs = (e0, e1)
    ibufs = (i0, i1)
    esems = (es0, es1)
    isems = (is0, is1)
    cid = lax.axis_index("c")
    sid = lax.axis_index("s")
    wid = sid * 2 + cid
    zf = jnp.zeros((16,), F32)

    def zr(r, carry):
        for k in range(D // 16):
            e0[r, pl.ds(k * 16, 16)] = zf
        return carry

    lax.fori_loop(0, 128, zr, 0)
    zlo = sid * TROWS
    for part in range(TROWS // 128):
        pltpu.sync_copy(e0, table.at[pl.ds(zlo + part * 128, 128)])
    plsc.subcore_barrier()

    u0 = wid * UPT + jnp.minimum(wid, UREM)
    nu = UPT + (wid < UREM).astype(I32)

    def issue(t, k):
        u = u0 + t
        pltpu.async_copy(he_hbm.at[pl.ds(u * 128, 128)], ebufs[k], esems[k])
        pltpu.async_copy(dst2d_hbm.at[pl.ds(u, 1)], ibufs[k], isems[k])

    def wait_u(t, k):
        u = u0 + t
        pltpu.make_async_copy(
            he_hbm.at[pl.ds(u * 128, 128)], ebufs[k], esems[k]
        ).wait()
        pltpu.make_async_copy(
            dst2d_hbm.at[pl.ds(u, 1)], ibufs[k], isems[k]
        ).wait()

    for k in range(2):
        @pl.when(k < nu)
        def _(k=k):
            issue(k, k)

    def grp(g, carry):
        for k in range(2):
            t = g * 2 + k

            @pl.when(t < nu)
            def _(t=t, k=k):
                wait_u(t, k)
                pltpu.sync_copy(ebufs[k], table.at[ibufs[k].at[0]], add=True)

                @pl.when(t + 2 < nu)
                def _():
                    issue(t + 2, k)

        return carry

    lax.fori_loop(0, (UPT + 2) // 2, grp, 0)
    plsc.subcore_barrier()
    pltpu.sync_copy(
        table.at[pl.ds(zlo, TROWS)], out_hbm.at[cid].at[pl.ds(zlo, TROWS)]
    )


NGC = EPT // CG


@functools.partial(
    pl.kernel,
    out_type=jax.ShapeDtypeStruct((E, D), F32),
    mesh=_mesh,
    compiler_params=pltpu.CompilerParams(needs_layout_passes=False),
    scratch_types=[
        pltpu.VMEM((CG,), I32),
        pltpu.VMEM((CG,), I32),
        pltpu.VMEM((CG, D), F32),
        pltpu.VMEM((CG, D), F32),
        pltpu.SemaphoreType.DMA,
        pltpu.SemaphoreType.DMA,
        pltpu.SemaphoreType.DMA,
        pltpu.SemaphoreType.DMA,
        pltpu.SemaphoreType.DMA,
        pltpu.SemaphoreType.DMA,
    ],
)
def _gather_rows(g_hbm, src_hbm, out_hbm, sb0, sb1, gb0, gb1,
                 ss0, ss1, gs0, gs1, os0, os1):
    sbufs = (sb0, sb1)
    gbufs = (gb0, gb1)
    ssems = (ss0, ss1)
    gsems = (gs0, gs1)
    osems = (os0, os1)
    wid = _wid()
    base = wid * EPT

    def issue_src(c, k):
        pltpu.async_copy(
            src_hbm.at[pl.ds(base + c * CG, CG)], sbufs[k], ssems[k]
        )

    def wait_src(c, k):
        pltpu.make_async_copy(
            src_hbm.at[pl.ds(base + c * CG, CG)], sbufs[k], ssems[k]
        ).wait()

    def issue_out(c, k):
        pltpu.async_copy(
            gbufs[k], out_hbm.at[pl.ds(base + c * CG, CG)], osems[k]
        )

    def wait_out(c, k):
        pltpu.make_async_copy(
            gbufs[k], out_hbm.at[pl.ds(base + c * CG, CG)], osems[k]
        ).wait()

    issue_src(0, 0)
    issue_src(1, 1)

    def chunk2(g, carry):
        for k in range(2):
            c = g * 2 + k
            wait_src(c, k)

            @pl.when(c >= 2)
            def _(c=c, k=k):
                wait_out(c - 2, k)

            cps = [
                pltpu.async_copy(
                    g_hbm.at[sbufs[k].at[pl.ds(j * 80, 80)]],
                    gbufs[k].at[pl.ds(j * 80, 80)],
                    gsems[k],
                )
                for j in range(CG // 80)
            ]
            for cp in cps:
                cp.wait()

            @pl.when(c + 2 < NGC)
            def _(c=c, k=k):
                issue_src(c + 2, k)

            issue_out(c, k)
        return carry

    lax.fori_loop(0, NGC // 2, chunk2, 0)
    if NGC % 2:
        c = NGC - 1
        wait_src(c, 0)
        wait_out(c - 2, 0)
        cps = [
            pltpu.async_copy(
                g_hbm.at[sbufs[0].at[pl.ds(j * 80, 80)]],
                gbufs[0].at[pl.ds(j * 80, 80)],
                gsems[0],
            )
            for j in range(CG // 80)
        ]
        for cp in cps:
            cp.wait()
        issue_out(c, 0)
    wait_out(NGC - 2, 1 if NGC % 2 else 0)
    wait_out(NGC - 1, 0 if NGC % 2 else 1)


def _relu_mm_body(x_ref, w_ref, o_ref):
    o_ref[...] = jnp.maximum(
        jnp.dot(x_ref[...], w_ref[...], preferred_element_type=F32), 0.0
    )


def _node_init(x, w):
    return pl.pallas_call(
        _relu_mm_body,
        out_shape=jax.ShapeDtypeStruct((N, D), F32),
    )(x, w)


def _edge_init(ea, w):
    blk = 8000
    return pl.pallas_call(
        _relu_mm_body,
        grid=(E // blk,),
        in_specs=[
            pl.BlockSpec((blk, ED), lambda i: (i, 0)),
            pl.BlockSpec((ED, D), lambda i: (0, 0)),
        ],
        out_specs=pl.BlockSpec((blk, D), lambda i: (i, 0)),
        out_shape=jax.ShapeDtypeStruct((E, D), F32),
    )(ea, w)


def _node_update_body(hn_ref, s_ref, m_ref, wc_ref, wh_ref, hn_out, g_out):
    msg = s_ref[...] * m_ref[...]
    hn = jnp.maximum(
        jnp.dot(hn_ref[...] + msg, wc_ref[...], preferred_element_type=F32), 0.0
    )
    hn_out[...] = hn
    g_out[...] = jnp.dot(hn, wh_ref[...], preferred_element_type=F32)


def _node_update(hn, s, m, wc, wh):
    return pl.pallas_call(
        _node_update_body,
        out_shape=(
            jax.ShapeDtypeStruct((N, D), F32),
            jax.ShapeDtypeStruct((N, D), F32),
        ),
    )(hn, s, m, wc, wh)


def _edge_premul_body(he_ref, he0_ref, wh_ref, o_ref):
    o_ref[...] = he0_ref[...] - jnp.dot(
        he_ref[...], wh_ref[...], preferred_element_type=F32
    )


def _edge_premul(he, he0, wh):
    blk = 8000
    return pl.pallas_call(
        _edge_premul_body,
        grid=(E // blk,),
        in_specs=[
            pl.BlockSpec((blk, D), lambda i: (i, 0)),
            pl.BlockSpec((blk, D), lambda i: (i, 0)),
            pl.BlockSpec((D, D), lambda i: (0, 0)),
        ],
        out_specs=pl.BlockSpec((blk, D), lambda i: (i, 0)),
        out_shape=jax.ShapeDtypeStruct((E, D), F32),
    )(he, he0, wh)


def _edge_update_body(he_ref, he0_ref, gs_ref, wh_ref, o_ref):
    o_ref[...] = jnp.maximum(
        he0_ref[...]
        - jnp.dot(he_ref[...], wh_ref[...], preferred_element_type=F32)
        + gs_ref[...],
        0.0,
    )


def _edge_update(he, he0, gs, wh):
    blk = 8000
    return pl.pallas_call(
        _edge_update_body,
        grid=(E // blk,),
        in_specs=[
            pl.BlockSpec((blk, D), lambda i: (i, 0)),
            pl.BlockSpec((blk, D), lambda i: (i, 0)),
            pl.BlockSpec((blk, D), lambda i: (i, 0)),
            pl.BlockSpec((D, D), lambda i: (0, 0)),
        ],
        out_specs=pl.BlockSpec((blk, D), lambda i: (i, 0)),
        out_shape=jax.ShapeDtypeStruct((E, D), F32),
    )(he, he0, gs, wh)


def _readout_body(hn_ref, agg_ref, b_ref, wo_ref, lw_ref, lb_ref, o_ref):
    wo = wo_ref[...]
    agg = agg_ref[0, :N] + agg_ref[1, :N]
    h_atom = jnp.maximum(
        jnp.dot(hn_ref[...], wo[:D], preferred_element_type=F32)
        + jnp.dot(agg, wo[D:], preferred_element_type=F32),
        0.0,
    )
    onehot = (b_ref[...] == lax.broadcasted_iota(I32, (1, G), 1)).astype(F32)
    sums = lax.dot_general(
        onehot, h_atom, (((0,), (0,)), ((), ())), preferred_element_type=F32
    )
    counts = jnp.sum(onehot, axis=0)[:, None]
    h_mol = sums / jnp.maximum(counts, 1.0)
    o_ref[...] = (
        jnp.dot(h_mol, lw_ref[...], preferred_element_type=F32) + lb_ref[...]
    )


def _readout(hn, agg, batch2d, wo, lw, lb2d):
    return pl.pallas_call(
        _readout_body,
        out_shape=jax.ShapeDtypeStruct((G, 1), F32),
    )(hn, agg, batch2d, wo, lw, lb2d)


@jax.jit
def kernel(x, edge_index, edge_attr, batch, W_i_node, W_i_edge, W_comm, W_h,
           W_o, lin_W, lin_b):
    src = edge_index[0]
    dst = edge_index[1]
    hn = _node_init(x, W_i_node)
    he0 = _edge_init(edge_attr, W_i_edge)
    he = he0
    for l in range(NLAYER):
        s_pad, m_pad = _seg_sum_max(he, dst)
        hn, g = _node_update(hn, s_pad[:N], m_pad[:N], W_comm[l], W_h[l])
        gs = _gather_rows(g, src)
        he = _edge_update(he, he0, gs, W_h[l])
    p = _seg_sum_atomic(he, dst.reshape(NUNIT, 128))
    return _readout(
        hn, p, batch.reshape(N, 1), W_o, lin_W, lin_b.reshape(1, 1)
    )
